# baseline (speedup 1.0000x reference)
import functools

import jax
import jax.numpy as jnp
from jax import lax
from jax.experimental import pallas as pl
from jax.experimental.pallas import tpu as pltpu
from jax.experimental.pallas import tpu_sc as plsc

BATCH = 16384
D = 32
LANES = 16
NC = 2
NS = 16
NW = NC * NS
BPW = BATCH // NW
NCHUNK = 4
CHUNK = BPW // NCHUNK
NGROUP = BPW // LANES


@functools.partial(
    pl.kernel,
    out_type=jax.ShapeDtypeStruct((BATCH,), jnp.float32),
    mesh=plsc.VectorSubcoreMesh(core_axis_name="c", subcore_axis_name="s"),
    compiler_params=pltpu.CompilerParams(
        needs_layout_passes=False, use_tc_tiling_on_sc=False
    ),
    scratch_types=[
        pltpu.VMEM((NCHUNK, CHUNK), jnp.int32),
        pltpu.VMEM((NCHUNK, CHUNK), jnp.int32),
        pltpu.VMEM((BPW, D), jnp.float32),
        pltpu.VMEM((BPW, D), jnp.float32),
        pltpu.VMEM((BPW,), jnp.float32),
        pltpu.SemaphoreType.DMA,
    ],
)
def _neumf_sc(users_hbm, items_hbm, ut_hbm, it_hbm, out_hbm,
              idx_u, idx_i, rows_u, rows_i, out_v, sem):
    wid = lax.axis_index("s") * NC + lax.axis_index("c")

    pltpu.sync_copy(users_hbm.at[wid], idx_u)
    pltpu.sync_copy(items_hbm.at[wid], idx_i)

    copies = []
    for j in range(NCHUNK):
        dst = pl.ds(j * CHUNK, CHUNK)
        copies.append(pltpu.async_copy(ut_hbm.at[idx_u.at[j]], rows_u.at[dst], sem))
        copies.append(pltpu.async_copy(it_hbm.at[idx_i.at[j]], rows_i.at[dst], sem))
    for cp in copies:
        cp.wait()

    lanes = lax.iota(jnp.int32, LANES)

    def group(g, carry):
        b_idx = g * LANES + lanes
        acc = jnp.zeros((LANES,), jnp.float32)
        for dd in range(D):
            d_idx = jnp.full((LANES,), dd, jnp.int32)
            u = plsc.load_gather(rows_u, [b_idx, d_idx])
            v = plsc.load_gather(rows_i, [b_idx, d_idx])
            acc = acc + u * v
        out_v[pl.ds(pl.multiple_of(g * LANES, LANES), LANES)] = acc
        return carry

    lax.fori_loop(0, NGROUP, group, 0)

    pltpu.sync_copy(out_v, out_hbm.at[pl.ds(wid * BPW, BPW)])


def kernel(users, items, user_table, item_table):
    u3 = users.reshape(NW, NCHUNK, CHUNK).astype(jnp.int32)
    i3 = items.reshape(NW, NCHUNK, CHUNK).astype(jnp.int32)
    return _neumf_sc(u3, i3, user_table, item_table)
